# Initial kernel scaffold; baseline (speedup 1.0000x reference)
import jax
import jax.numpy as jnp
from jax.experimental import pallas as pl


def kernel(x, w0, w1, w2, w3, w4, w5, b0, b1, b2, b3, b4, b5):
    raise NotImplementedError("write your pallas kernel here")


# speedup vs baseline: 17.6707x; 17.6707x over previous
import functools

import jax
import jax.numpy as jnp
from jax import lax
from jax.experimental import pallas as pl
from jax.experimental.pallas import tpu as pltpu

NEG_SLOPE = 0.2
K_NEIGHBORS = 32
MAX_ROWS = 4096


def _lrelu(z):
    return jnp.maximum(z, NEG_SLOPE * z)


def _chunk_slices(k, n):
    s = max(1, min(k, MAX_ROWS // max(n, 1)))
    while k % s:
        s -= 1
    return s


def _dgcnn_kernel(x_ref, xt_ref, xbt_ref,
                  w1a, w1b, w2, w3, w4, w5,
                  w6a, w6b, w6c, w6d, w6e, w6f,
                  b1, b2, b3, b4, b5, b6,
                  out_ref, pd_ref, g_ref,
                  *, n, k, sl):
    nchunk = k // sl

    x = x_ref[0]
    xt = xt_ref[0]
    inner = jnp.dot(x, xt, preferred_element_type=jnp.float32)
    sq_col = jnp.sum(x * x, axis=-1, keepdims=True)
    sq_row = jnp.sum(xt * xt, axis=0, keepdims=True)
    pd_ref[...] = 2.0 * inner - sq_col - sq_row

    xbt = xbt_ref[0]
    q = jnp.dot(w1b[...], xbt,
                preferred_element_type=jnp.float32) + b1[...]
    q8 = jnp.concatenate([q] * sl, axis=1)

    iota0 = lax.broadcasted_iota(jnp.int32, (n, n), 0)
    minus_inf = jnp.float32(-jnp.inf)

    acc = [None] * 5

    def fold(a, h):
        m = h[:, 0:n]
        for i in range(1, sl):
            m = jnp.maximum(m, h[:, i * n:(i + 1) * n])
        return m if a is None else jnp.maximum(a, m)

    for c in range(nchunk):
        def extract(t, _):
            pd = pd_ref[...]
            am = jnp.argmax(pd, axis=0, keepdims=True)
            onehot = iota0 == am
            g_ref[t] = jnp.where(onehot, 1.0, 0.0).astype(jnp.bfloat16)
            pd_ref[...] = jnp.where(onehot, minus_inf, pd)
            return 0

        lax.fori_loop(0, sl, extract, 0)

        nbt = jnp.concatenate(
            [jnp.dot(xbt, g_ref[t], preferred_element_type=jnp.float32)
             for t in range(sl)], axis=1)
        nbt = nbt.astype(jnp.bfloat16)

        h1 = _lrelu(jnp.dot(w1a[...], nbt,
                            preferred_element_type=jnp.float32) + q8)
        acc[0] = fold(acc[0], h1)
        h2 = _lrelu(jnp.dot(w2[...], h1.astype(jnp.bfloat16),
                            preferred_element_type=jnp.float32) + b2[...])
        acc[1] = fold(acc[1], h2)
        h3 = _lrelu(jnp.dot(w3[...], h2.astype(jnp.bfloat16),
                            preferred_element_type=jnp.float32) + b3[...])
        acc[2] = fold(acc[2], h3)
        h4 = _lrelu(jnp.dot(w4[...], h3.astype(jnp.bfloat16),
                            preferred_element_type=jnp.float32) + b4[...])
        acc[3] = fold(acc[3], h4)
        h5 = _lrelu(jnp.dot(w5[...], h4.astype(jnp.bfloat16),
                            preferred_element_type=jnp.float32) + b5[...])
        acc[4] = fold(acc[4], h5)

    x1 = acc[0].astype(jnp.bfloat16)
    x2 = acc[1].astype(jnp.bfloat16)
    x3 = acc[2].astype(jnp.bfloat16)
    x4 = acc[3].astype(jnp.bfloat16)
    x5f = acc[4]
    x5 = x5f.astype(jnp.bfloat16)
    g = jnp.mean(x5f, axis=1, keepdims=True).astype(jnp.bfloat16)
    z = (jnp.dot(w6a[...], x1, preferred_element_type=jnp.float32)
         + jnp.dot(w6b[...], x2, preferred_element_type=jnp.float32)
         + jnp.dot(w6c[...], x3, preferred_element_type=jnp.float32)
         + jnp.dot(w6d[...], x4, preferred_element_type=jnp.float32)
         + jnp.dot(w6e[...], x5, preferred_element_type=jnp.float32)
         + jnp.dot(w6f[...], g, preferred_element_type=jnp.float32)
         + b6[...])
    out_ref[0] = _lrelu(z).T


@jax.jit
def kernel(x, w0, w1, w2, w3, w4, w5, b0, b1, b2, b3, b4, b5):
    B, N, _ = x.shape
    emb = w5.shape[1]
    k = K_NEIGHBORS
    sl = _chunk_slices(k, N)

    xt = jnp.transpose(x, (0, 2, 1))
    xbt = xt.astype(jnp.bfloat16)
    w1a = w0[0:3].T.astype(jnp.bfloat16)
    w1b = w0[3:6].T.astype(jnp.bfloat16)
    wr = [w.T.astype(jnp.bfloat16) for w in (w1, w2, w3, w4)]
    w6s = [w5[a:b].T.astype(jnp.bfloat16)
           for a, b in ((0, 64), (64, 128), (128, 256),
                        (256, 512), (512, 1024), (1024, 1536))]
    bs = [bb.T for bb in (b0, b1, b2, b3, b4, b5)]

    wmats = [w1a, w1b, *wr, *w6s]
    wspecs = [pl.BlockSpec(w.shape, lambda b: (0, 0)) for w in wmats]
    bspecs = [pl.BlockSpec(bb.shape, lambda b: (0, 0)) for bb in bs]

    body = functools.partial(_dgcnn_kernel, n=N, k=k, sl=sl)
    return pl.pallas_call(
        body,
        out_shape=jax.ShapeDtypeStruct((B, N, emb), jnp.float32),
        grid=(B,),
        in_specs=[pl.BlockSpec((1, N, 3), lambda b: (b, 0, 0)),
                  pl.BlockSpec((1, 3, N), lambda b: (b, 0, 0)),
                  pl.BlockSpec((1, 3, N), lambda b: (b, 0, 0))]
                 + wspecs + bspecs,
        out_specs=pl.BlockSpec((1, N, emb), lambda b: (b, 0, 0)),
        scratch_shapes=[pltpu.VMEM((N, N), jnp.float32),
                        pltpu.VMEM((sl, N, N), jnp.bfloat16)],
        compiler_params=pltpu.CompilerParams(
            dimension_semantics=("parallel",),
            vmem_limit_bytes=48 * 1024 * 1024),
    )(x, xt, xbt, *wmats, *bs)
